# Initial kernel scaffold; baseline (speedup 1.0000x reference)
import jax
import jax.numpy as jnp
from jax.experimental import pallas as pl


def kernel(x, edge_index, edge_weight, batch, W1_rel, b1, W1_root, W2_rel, b2, W2_root, W3_rel, b3, W3_root, gamma1, beta1, gamma2, beta2, Wlin, blin):
    raise NotImplementedError("write your pallas kernel here")


# speedup vs baseline: 3.5970x; 3.5970x over previous
import functools

import jax
import jax.numpy as jnp
from jax import lax
from jax.experimental import pallas as pl
from jax.experimental.pallas import tpu as pltpu
from jax.experimental.pallas import tpu_sc as plsc

N = 10000
D = 128
E = 320000
G = 64

NC = 2
NS = 16
NW = NC * NS
EPT = E // NW
CHUNK = 80
NCHUNK = EPT // CHUNK
WB_CHUNK = 80
NWB = N // WB_CHUNK
WB_ROUNDS = (NWB + NS - 1) // NS
FB = D // 16


def _sc_segsum_body(x_hbm, src_hbm, dst_hbm, w16_hbm, zeros_hbm, out_hbm,
                    acc_sh, src_v, dst_v, w16_v, rows_v, sem):
    c = lax.axis_index("c")
    s = lax.axis_index("s")
    wid = c * NS + s

    pltpu.sync_copy(src_hbm.at[wid], src_v)
    pltpu.sync_copy(dst_hbm.at[wid], dst_v)

    for k in range(WB_ROUNDS):
        cidx = s + k * NS

        @pl.when(cidx < NWB)
        def _():
            pltpu.sync_copy(zeros_hbm.at[pl.ds(cidx * WB_CHUNK, WB_CHUNK)],
                            acc_sh.at[pl.ds(cidx * WB_CHUNK, WB_CHUNK)])
    plsc.subcore_barrier()

    def chunk_body(j, carry):
        pltpu.async_copy(x_hbm.at[src_v.at[j]], rows_v, sem).wait()
        pltpu.sync_copy(w16_hbm.at[wid, j], w16_v)
        for e in range(CHUNK):
            wvec = w16_v[pl.ds(e * 16, 16)]
            for f in range(FB):
                rows_v[e, pl.ds(f * 16, 16)] = rows_v[e, pl.ds(f * 16, 16)] * wvec
        pltpu.sync_copy(rows_v, acc_sh.at[dst_v.at[j]], add=True)
        return carry

    lax.fori_loop(0, NCHUNK, chunk_body, 0)

    plsc.subcore_barrier()
    for k in range(WB_ROUNDS):
        cidx = s + k * NS

        @pl.when(cidx < NWB)
        def _():
            base = cidx * WB_CHUNK
            pltpu.sync_copy(acc_sh.at[pl.ds(base, WB_CHUNK)], rows_v)
            pltpu.sync_copy(rows_v, out_hbm.at[c, pl.ds(base, WB_CHUNK)])


def _make_segsum(interpret=False):
    mesh = plsc.VectorSubcoreMesh(core_axis_name="c", subcore_axis_name="s",
                                  num_cores=NC, num_subcores=NS)
    return pl.kernel(
        _sc_segsum_body,
        out_type=jax.ShapeDtypeStruct((NC, N, D), jnp.float32),
        mesh=mesh,
        scratch_types=[
            pltpu.VMEM_SHARED((N, D), jnp.float32),
            pltpu.VMEM((NCHUNK, CHUNK), jnp.int32),
            pltpu.VMEM((NCHUNK, CHUNK), jnp.int32),
            pltpu.VMEM((CHUNK * 16,), jnp.float32),
            pltpu.VMEM((CHUNK, D), jnp.float32),
            pltpu.SemaphoreType.DMA,
        ],
        interpret=interpret,
    )


def _tc_layer_body(p_ref, h_ref, wrelT_ref, wrootT_ref, b_ref,
                   gamma_ref, beta_ref, out_ref):
    agg = p_ref[0] + p_ref[1]
    y = (jnp.dot(agg, wrelT_ref[...], preferred_element_type=jnp.float32)
         + jnp.dot(h_ref[...], wrootT_ref[...], preferred_element_type=jnp.float32)
         + b_ref[...])
    mean = jnp.mean(y, axis=0, keepdims=True)
    var = jnp.mean(jnp.square(y - mean), axis=0, keepdims=True)
    yn = (y - mean) * lax.rsqrt(var + 1e-5) * gamma_ref[...] + beta_ref[...]
    out_ref[...] = jnp.maximum(yn, 0.0)


def _make_tc_layer(interpret=False):
    return pl.pallas_call(
        _tc_layer_body,
        out_shape=jax.ShapeDtypeStruct((N, D), jnp.float32),
        interpret=interpret,
    )


def _tc_final_body(p_ref, h_ref, batch_ref, wrelT_ref, wrootT_ref, b_ref,
                   wlinT_ref, blin_ref, out_ref):
    agg = p_ref[0] + p_ref[1]
    y = (jnp.dot(agg, wrelT_ref[...], preferred_element_type=jnp.float32)
         + jnp.dot(h_ref[...], wrootT_ref[...], preferred_element_type=jnp.float32)
         + b_ref[...])
    gid = lax.broadcasted_iota(jnp.int32, (G, N), 0)
    sel = (batch_ref[...] == gid).astype(jnp.float32)
    sums = jnp.dot(sel, y, preferred_element_type=jnp.float32)
    counts = jnp.sum(sel, axis=1, keepdims=True)
    pooled = sums / jnp.maximum(counts, 1.0)
    out_ref[...] = (jnp.dot(pooled, wlinT_ref[...],
                            preferred_element_type=jnp.float32)
                    + blin_ref[...])


def _make_tc_final(interpret=False):
    return pl.pallas_call(
        _tc_final_body,
        out_shape=jax.ShapeDtypeStruct((G, 2), jnp.float32),
        interpret=interpret,
    )


@jax.jit
def kernel(x, edge_index, edge_weight, batch,
           W1_rel, b1, W1_root, W2_rel, b2, W2_root, W3_rel, b3, W3_root,
           gamma1, beta1, gamma2, beta2, Wlin, blin):
    src = edge_index[0].reshape(NW, NCHUNK, CHUNK)
    dst = edge_index[1].reshape(NW, NCHUNK, CHUNK)
    w16 = jnp.repeat(edge_weight[:, None], 16, axis=1).reshape(
        NW, NCHUNK, CHUNK * 16)
    zeros = jnp.zeros((N, D), jnp.float32)

    segsum = _make_segsum()
    tc_layer = _make_tc_layer()
    tc_final = _make_tc_final()

    p1 = segsum(x, src, dst, w16, zeros)
    h1 = tc_layer(p1, x, W1_rel.T, W1_root.T, b1[None],
                  gamma1[None], beta1[None])
    p2 = segsum(h1, src, dst, w16, zeros)
    h2 = tc_layer(p2, h1, W2_rel.T, W2_root.T, b2[None],
                  gamma2[None], beta2[None])
    p3 = segsum(h2, src, dst, w16, zeros)
    return tc_final(p3, h2, batch[None], W3_rel.T, W3_root.T, b3[None],
                    Wlin.T, blin[None])


# speedup vs baseline: 5.4929x; 1.5271x over previous
import jax
import jax.numpy as jnp
from jax import lax
from jax.experimental import pallas as pl
from jax.experimental.pallas import tpu as pltpu
from jax.experimental.pallas import tpu_sc as plsc

N = 10000
D = 128
E = 320000
G = 64

NC = 2
NS = 16
NW = NC * NS
EPT = E // NW
CHUNK = 80
NCHUNK = EPT // CHUNK
CROWS = 18
WB_CHUNK = 80
NWB = N // WB_CHUNK
WB_ROUNDS = (NWB + NS - 1) // NS
FB = D // 16
NBUF = 3
LOOP_SLOTS = NCHUNK - 2


def _scale(rows_v, w_v):
    def body(eb, carry):
        col = pl.multiple_of(eb * 16, 16)
        for k in range(16):
            e = eb * 16 + k
            wvec = w_v[k, pl.ds(col, 16)]
            for f in range(FB):
                rows_v[e, pl.ds(f * 16, 16)] = (
                    rows_v[e, pl.ds(f * 16, 16)] * wvec)
        return carry
    lax.fori_loop(0, CHUNK // 16, body, 0)


def _sc_segsum_body(x_hbm, idx_hbm, w16_hbm, zeros_hbm, out_hbm,
                    acc_sh, rows0, rows1, rows2, cb0, cb1, cb2,
                    wb0, wb1, wb2, g0, g1, g2, s0, s1, s2):
    c = lax.axis_index("c")
    s = lax.axis_index("s")
    wid = c * NS + s
    rows = (rows0, rows1, rows2)
    cbs = (cb0, cb1, cb2)
    wbs = (wb0, wb1, wb2)
    gsems = (g0, g1, g2)
    ssems = (s0, s1, s2)

    for k in range(WB_ROUNDS):
        cidx = s + k * NS

        @pl.when(cidx < NWB)
        def _():
            pltpu.sync_copy(zeros_hbm.at[pl.ds(cidx * WB_CHUNK, WB_CHUNK)],
                            acc_sh.at[pl.ds(cidx * WB_CHUNK, WB_CHUNK)])
    plsc.subcore_barrier()

    def issue_gather(buf, chunk_idx):
        pltpu.sync_copy(idx_hbm.at[wid, chunk_idx], cbs[buf])
        pltpu.sync_copy(w16_hbm.at[wid, chunk_idx], wbs[buf])
        pltpu.async_copy(x_hbm.at[cbs[buf].at[0]], rows[buf], gsems[buf])

    def wait_gather(buf):
        pltpu.make_async_copy(x_hbm.at[cbs[buf].at[0]], rows[buf],
                              gsems[buf]).wait()

    def issue_scatter(buf):
        pltpu.async_copy(rows[buf], acc_sh.at[cbs[buf].at[1]], ssems[buf],
                         add=True)

    def wait_scatter(buf):
        pltpu.make_async_copy(rows[buf], acc_sh.at[cbs[buf].at[1]],
                              ssems[buf]).wait()

    issue_gather(0, 0)
    issue_gather(1, 1)

    def slot_body(g, carry):
        for off in range(NBUF):
            cidx = 3 * g + off
            nbuf = (off + 2) % 3
            wait_gather(off)
            _scale(rows[off], wbs[off])
            issue_scatter(off)

            @pl.when(cidx >= 1)
            def _():
                wait_scatter(nbuf)
            issue_gather(nbuf, cidx + 2)
        return carry

    lax.fori_loop(0, LOOP_SLOTS // NBUF, slot_body, 0)

    for buf in (0, 1):
        wait_gather(buf)
        _scale(rows[buf], wbs[buf])
        issue_scatter(buf)
    wait_scatter(2)
    wait_scatter(0)
    wait_scatter(1)

    plsc.subcore_barrier()
    for k in range(WB_ROUNDS):
        cidx = s + k * NS

        @pl.when(cidx < NWB)
        def _():
            base = cidx * WB_CHUNK
            pltpu.sync_copy(acc_sh.at[pl.ds(base, WB_CHUNK)], rows0)
            pltpu.sync_copy(rows0, out_hbm.at[c, pl.ds(base, WB_CHUNK)])


def _make_segsum(interpret=False):
    mesh = plsc.VectorSubcoreMesh(core_axis_name="c", subcore_axis_name="s",
                                  num_cores=NC, num_subcores=NS)
    return pl.kernel(
        _sc_segsum_body,
        out_type=jax.ShapeDtypeStruct((NC, N, D), jnp.float32),
        mesh=mesh,
        scratch_types=(
            [pltpu.VMEM_SHARED((N, D), jnp.float32)]
            + [pltpu.VMEM((CHUNK, D), jnp.float32) for _ in range(NBUF)]
            + [pltpu.VMEM((2, CHUNK), jnp.int32) for _ in range(NBUF)]
            + [pltpu.VMEM((16, CHUNK), jnp.float32) for _ in range(NBUF)]
            + [pltpu.SemaphoreType.DMA for _ in range(2 * NBUF)]
        ),
        interpret=interpret,
    )


def _make_combo(edge_index, edge_weight):
    src = edge_index[0].reshape(NW, NCHUNK, 1, CHUNK)
    dst = edge_index[1].reshape(NW, NCHUNK, 1, CHUNK)
    idx = jnp.concatenate([src, dst], axis=2)
    w5 = edge_weight.reshape(NW, NCHUNK, CHUNK // 16, 16)
    wt = jnp.transpose(w5, (0, 1, 3, 2))[..., None]
    w16 = jnp.broadcast_to(
        wt, (NW, NCHUNK, 16, CHUNK // 16, 16)).reshape(NW, NCHUNK, 16, CHUNK)
    return idx, w16


def _tc_layer_body(p_ref, h_ref, wrelT_ref, wrootT_ref, b_ref,
                   gamma_ref, beta_ref, out_ref):
    agg = p_ref[0] + p_ref[1]
    y = (jnp.dot(agg, wrelT_ref[...], preferred_element_type=jnp.float32)
         + jnp.dot(h_ref[...], wrootT_ref[...], preferred_element_type=jnp.float32)
         + b_ref[...])
    mean = jnp.mean(y, axis=0, keepdims=True)
    var = jnp.mean(jnp.square(y - mean), axis=0, keepdims=True)
    yn = (y - mean) * lax.rsqrt(var + 1e-5) * gamma_ref[...] + beta_ref[...]
    out_ref[...] = jnp.maximum(yn, 0.0)


def _make_tc_layer(interpret=False):
    return pl.pallas_call(
        _tc_layer_body,
        out_shape=jax.ShapeDtypeStruct((N, D), jnp.float32),
        interpret=interpret,
    )


def _tc_final_body(p_ref, h_ref, batch_ref, wrelT_ref, wrootT_ref, b_ref,
                   wlinT_ref, blin_ref, out_ref):
    agg = p_ref[0] + p_ref[1]
    y = (jnp.dot(agg, wrelT_ref[...], preferred_element_type=jnp.float32)
         + jnp.dot(h_ref[...], wrootT_ref[...], preferred_element_type=jnp.float32)
         + b_ref[...])
    gid = lax.broadcasted_iota(jnp.int32, (G, N), 0)
    sel = (batch_ref[...] == gid).astype(jnp.float32)
    sums = jnp.dot(sel, y, preferred_element_type=jnp.float32)
    counts = jnp.sum(sel, axis=1, keepdims=True)
    pooled = sums / jnp.maximum(counts, 1.0)
    out_ref[...] = (jnp.dot(pooled, wlinT_ref[...],
                            preferred_element_type=jnp.float32)
                    + blin_ref[...])


def _make_tc_final(interpret=False):
    return pl.pallas_call(
        _tc_final_body,
        out_shape=jax.ShapeDtypeStruct((G, 2), jnp.float32),
        interpret=interpret,
    )


@jax.jit
def kernel(x, edge_index, edge_weight, batch,
           W1_rel, b1, W1_root, W2_rel, b2, W2_root, W3_rel, b3, W3_root,
           gamma1, beta1, gamma2, beta2, Wlin, blin):
    idx, w16 = _make_combo(edge_index, edge_weight)
    zeros = jnp.zeros((N, D), jnp.float32)

    segsum = _make_segsum()
    tc_layer = _make_tc_layer()
    tc_final = _make_tc_final()

    p1 = segsum(x, idx, w16, zeros)
    h1 = tc_layer(p1, x, W1_rel.T, W1_root.T, b1[None],
                  gamma1[None], beta1[None])
    p2 = segsum(h1, idx, w16, zeros)
    h2 = tc_layer(p2, h1, W2_rel.T, W2_root.T, b2[None],
                  gamma2[None], beta2[None])
    p3 = segsum(h2, idx, w16, zeros)
    return tc_final(p3, h2, batch[None], W3_rel.T, W3_root.T, b3[None],
                    Wlin.T, blin[None])


# speedup vs baseline: 6.1062x; 1.1117x over previous
import jax
import jax.numpy as jnp
from jax import lax
from jax.experimental import pallas as pl
from jax.experimental.pallas import tpu as pltpu
from jax.experimental.pallas import tpu_sc as plsc

N = 10000
D = 128
E = 320000
G = 64

NC = 2
NS = 16
NW = NC * NS
EPT = E // NW
CHUNK = 80
NCHUNK = EPT // CHUNK
CROWS = 18
WB_CHUNK = 80
NWB = N // WB_CHUNK
WB_ROUNDS = (NWB + NS - 1) // NS
FB = D // 16
NBUF = 3
LOOP_SLOTS = NCHUNK - 2


def _scale(rows_v, w_v):
    def body(eb, carry):
        col = pl.multiple_of(eb * 16, 16)
        for k in range(16):
            e = eb * 16 + k
            wvec = w_v[k, pl.ds(col, 16)]
            for f in range(FB):
                rows_v[e, pl.ds(f * 16, 16)] = (
                    rows_v[e, pl.ds(f * 16, 16)] * wvec)
        return carry
    lax.fori_loop(0, CHUNK // 16, body, 0)


def _sc_segsum_body(x_hbm, idx_hbm, w16_hbm, zeros_hbm, out_hbm,
                    acc_sh, rows0, rows1, rows2, cb0, cb1, cb2,
                    wb0, wb1, wb2, g0, g1, g2, s0, s1, s2,
                    ws0, ws1, ws2):
    c = lax.axis_index("c")
    s = lax.axis_index("s")
    wid = c * NS + s
    rows = (rows0, rows1, rows2)
    cbs = (cb0, cb1, cb2)
    wbs = (wb0, wb1, wb2)
    gsems = (g0, g1, g2)
    ssems = (s0, s1, s2)
    wsems = (ws0, ws1, ws2)

    for k in range(WB_ROUNDS):
        cidx = s + k * NS

        @pl.when(cidx < NWB)
        def _():
            pltpu.sync_copy(zeros_hbm.at[pl.ds(cidx * WB_CHUNK, WB_CHUNK)],
                            acc_sh.at[pl.ds(cidx * WB_CHUNK, WB_CHUNK)])
    plsc.subcore_barrier()

    def issue_gather(buf, chunk_idx):
        pltpu.sync_copy(idx_hbm.at[wid, chunk_idx], cbs[buf])
        pltpu.async_copy(x_hbm.at[cbs[buf].at[0]], rows[buf], gsems[buf])
        pltpu.async_copy(w16_hbm.at[wid, chunk_idx], wbs[buf], wsems[buf])

    def wait_gather(buf):
        pltpu.make_async_copy(x_hbm.at[cbs[buf].at[0]], rows[buf],
                              gsems[buf]).wait()

    def wait_weights(buf, chunk_idx):
        pltpu.make_async_copy(w16_hbm.at[wid, chunk_idx], wbs[buf],
                              wsems[buf]).wait()

    def issue_scatter(buf):
        pltpu.async_copy(rows[buf], acc_sh.at[cbs[buf].at[1]], ssems[buf],
                         add=True)

    def wait_scatter(buf):
        pltpu.make_async_copy(rows[buf], acc_sh.at[cbs[buf].at[1]],
                              ssems[buf]).wait()

    issue_gather(0, 0)
    issue_gather(1, 1)

    def slot_body(g, carry):
        for off in range(NBUF):
            cidx = 3 * g + off
            nbuf = (off + 2) % 3
            wait_gather(off)

            @pl.when(cidx >= 1)
            def _():
                wait_scatter(nbuf)
            issue_gather(nbuf, cidx + 2)
            wait_weights(off, cidx)
            _scale(rows[off], wbs[off])
            issue_scatter(off)
        return carry

    lax.fori_loop(0, LOOP_SLOTS // NBUF, slot_body, 0)

    wait_gather(0)
    wait_weights(0, NCHUNK - 2)
    _scale(rows[0], wbs[0])
    issue_scatter(0)

    wait_gather(1)
    wait_weights(1, NCHUNK - 1)
    _scale(rows[1], wbs[1])
    issue_scatter(1)

    wait_scatter(2)
    wait_scatter(0)
    wait_scatter(1)

    plsc.subcore_barrier()
    for k in range(WB_ROUNDS):
        cidx = s + k * NS

        @pl.when(cidx < NWB)
        def _():
            base = cidx * WB_CHUNK
            pltpu.sync_copy(acc_sh.at[pl.ds(base, WB_CHUNK)], rows0)
            pltpu.sync_copy(rows0, out_hbm.at[c, pl.ds(base, WB_CHUNK)])


def _make_segsum(interpret=False):
    mesh = plsc.VectorSubcoreMesh(core_axis_name="c", subcore_axis_name="s",
                                  num_cores=NC, num_subcores=NS)
    return pl.kernel(
        _sc_segsum_body,
        out_type=jax.ShapeDtypeStruct((NC, N, D), jnp.float32),
        mesh=mesh,
        scratch_types=(
            [pltpu.VMEM_SHARED((N, D), jnp.float32)]
            + [pltpu.VMEM((CHUNK, D), jnp.float32) for _ in range(NBUF)]
            + [pltpu.VMEM((2, CHUNK), jnp.int32) for _ in range(NBUF)]
            + [pltpu.VMEM((16, CHUNK), jnp.float32) for _ in range(NBUF)]
            + [pltpu.SemaphoreType.DMA for _ in range(3 * NBUF)]
        ),
        interpret=interpret,
    )


def _make_combo(edge_index, edge_weight):
    src = edge_index[0].reshape(NW, NCHUNK, 1, CHUNK)
    dst = edge_index[1].reshape(NW, NCHUNK, 1, CHUNK)
    idx = jnp.concatenate([src, dst], axis=2)
    w5 = edge_weight.reshape(NW, NCHUNK, CHUNK // 16, 16)
    wt = jnp.transpose(w5, (0, 1, 3, 2))[..., None]
    w16 = jnp.broadcast_to(
        wt, (NW, NCHUNK, 16, CHUNK // 16, 16)).reshape(NW, NCHUNK, 16, CHUNK)
    return idx, w16


def _tc_layer_body(p_ref, h_ref, wrelT_ref, wrootT_ref, b_ref,
                   gamma_ref, beta_ref, out_ref):
    agg = p_ref[0] + p_ref[1]
    y = (jnp.dot(agg, wrelT_ref[...], preferred_element_type=jnp.float32)
         + jnp.dot(h_ref[...], wrootT_ref[...], preferred_element_type=jnp.float32)
         + b_ref[...])
    mean = jnp.mean(y, axis=0, keepdims=True)
    var = jnp.mean(jnp.square(y - mean), axis=0, keepdims=True)
    yn = (y - mean) * lax.rsqrt(var + 1e-5) * gamma_ref[...] + beta_ref[...]
    out_ref[...] = jnp.maximum(yn, 0.0)


def _make_tc_layer(interpret=False):
    return pl.pallas_call(
        _tc_layer_body,
        out_shape=jax.ShapeDtypeStruct((N, D), jnp.float32),
        interpret=interpret,
    )


def _tc_final_body(p_ref, h_ref, batch_ref, wrelT_ref, wrootT_ref, b_ref,
                   wlinT_ref, blin_ref, out_ref):
    agg = p_ref[0] + p_ref[1]
    y = (jnp.dot(agg, wrelT_ref[...], preferred_element_type=jnp.float32)
         + jnp.dot(h_ref[...], wrootT_ref[...], preferred_element_type=jnp.float32)
         + b_ref[...])
    gid = lax.broadcasted_iota(jnp.int32, (G, N), 0)
    sel = (batch_ref[...] == gid).astype(jnp.float32)
    sums = jnp.dot(sel, y, preferred_element_type=jnp.float32)
    counts = jnp.sum(sel, axis=1, keepdims=True)
    pooled = sums / jnp.maximum(counts, 1.0)
    out_ref[...] = (jnp.dot(pooled, wlinT_ref[...],
                            preferred_element_type=jnp.float32)
                    + blin_ref[...])


def _make_tc_final(interpret=False):
    return pl.pallas_call(
        _tc_final_body,
        out_shape=jax.ShapeDtypeStruct((G, 2), jnp.float32),
        interpret=interpret,
    )


@jax.jit
def kernel(x, edge_index, edge_weight, batch,
           W1_rel, b1, W1_root, W2_rel, b2, W2_root, W3_rel, b3, W3_root,
           gamma1, beta1, gamma2, beta2, Wlin, blin):
    idx, w16 = _make_combo(edge_index, edge_weight)
    zeros = jnp.zeros((N, D), jnp.float32)

    segsum = _make_segsum()
    tc_layer = _make_tc_layer()
    tc_final = _make_tc_final()

    p1 = segsum(x, idx, w16, zeros)
    h1 = tc_layer(p1, x, W1_rel.T, W1_root.T, b1[None],
                  gamma1[None], beta1[None])
    p2 = segsum(h1, idx, w16, zeros)
    h2 = tc_layer(p2, h1, W2_rel.T, W2_root.T, b2[None],
                  gamma2[None], beta2[None])
    p3 = segsum(h2, idx, w16, zeros)
    return tc_final(p3, h2, batch[None], W3_rel.T, W3_root.T, b3[None],
                    Wlin.T, blin[None])


# speedup vs baseline: 8.3386x; 1.3656x over previous
import jax
import jax.numpy as jnp
from jax import lax
from jax.experimental import pallas as pl
from jax.experimental.pallas import tpu as pltpu
from jax.experimental.pallas import tpu_sc as plsc

N = 10000
D = 128
E = 320000
G = 64

NC = 2
NS = 16
NW = NC * NS
EPT = E // NW
CHUNK = 80
NCHUNK = EPT // CHUNK
CROWS = 18
WB_CHUNK = 80
NWB = N // WB_CHUNK
WB_ROUNDS = (NWB + NS - 1) // NS
FB = D // 16
NBUF = 3
LOOP_SLOTS = NCHUNK - 2


def _scale(rows_v, w_v):
    def body(eb, carry):
        col = pl.multiple_of(eb * 16, 16)
        for k in range(16):
            e = eb * 16 + k
            wvec = w_v[k, pl.ds(col, 16)]
            for f in range(FB):
                rows_v[e, pl.ds(f * 16, 16)] = (
                    rows_v[e, pl.ds(f * 16, 16)] * wvec)
        return carry
    lax.fori_loop(0, CHUNK // 16, body, 0)


def _sc_segsum_body(x_hbm, src_hbm, dst_hbm, w16_hbm, zeros_hbm, out_hbm,
                    acc_sh, rows0, rows1, rows2, sb0, sb1, sb2,
                    db0, db1, db2, wb0, wb1, wb2,
                    g0, g1, g2, s0, s1, s2, ws0, ws1, ws2,
                    is0, is1, is2, id0, id1, id2):
    c = lax.axis_index("c")
    s = lax.axis_index("s")
    wid = c * NS + s
    rows = (rows0, rows1, rows2)
    sbs = (sb0, sb1, sb2)
    dbs = (db0, db1, db2)
    wbs = (wb0, wb1, wb2)
    gsems = (g0, g1, g2)
    ssems = (s0, s1, s2)
    wsems = (ws0, ws1, ws2)
    srcsems = (is0, is1, is2)
    dstsems = (id0, id1, id2)

    for k in range(WB_ROUNDS):
        cidx = s + k * NS

        @pl.when(cidx < NWB)
        def _():
            pltpu.sync_copy(zeros_hbm.at[pl.ds(cidx * WB_CHUNK, WB_CHUNK)],
                            acc_sh.at[pl.ds(cidx * WB_CHUNK, WB_CHUNK)])
    plsc.subcore_barrier()

    def issue_src(buf, chunk_idx):
        pltpu.async_copy(src_hbm.at[wid, chunk_idx], sbs[buf], srcsems[buf])

    def wait_src(buf, chunk_idx):
        pltpu.make_async_copy(src_hbm.at[wid, chunk_idx], sbs[buf],
                              srcsems[buf]).wait()

    def issue_dst(buf, chunk_idx):
        pltpu.async_copy(dst_hbm.at[wid, chunk_idx], dbs[buf], dstsems[buf])

    def wait_dst(buf, chunk_idx):
        pltpu.make_async_copy(dst_hbm.at[wid, chunk_idx], dbs[buf],
                              dstsems[buf]).wait()

    def issue_weights(buf, chunk_idx):
        pltpu.async_copy(w16_hbm.at[wid, chunk_idx], wbs[buf], wsems[buf])

    def wait_weights(buf, chunk_idx):
        pltpu.make_async_copy(w16_hbm.at[wid, chunk_idx], wbs[buf],
                              wsems[buf]).wait()

    def issue_gather(buf):
        pltpu.async_copy(x_hbm.at[sbs[buf].at[0]], rows[buf], gsems[buf])

    def wait_gather(buf):
        pltpu.make_async_copy(x_hbm.at[sbs[buf].at[0]], rows[buf],
                              gsems[buf]).wait()

    def issue_scatter(buf):
        pltpu.async_copy(rows[buf], acc_sh.at[dbs[buf].at[0]], ssems[buf],
                         add=True)

    def wait_scatter(buf):
        pltpu.make_async_copy(rows[buf], acc_sh.at[dbs[buf].at[0]],
                              ssems[buf]).wait()

    pltpu.sync_copy(src_hbm.at[wid, 0], sbs[0])
    issue_gather(0)
    issue_weights(0, 0)
    issue_dst(0, 0)
    pltpu.sync_copy(src_hbm.at[wid, 1], sbs[1])
    issue_gather(1)
    issue_weights(1, 1)
    issue_dst(1, 1)
    issue_src(2, 2)

    def slot_body(g, carry):
        for off in range(NBUF):
            cidx = 3 * g + off
            nbuf = (off + 2) % 3
            wait_gather(off)

            @pl.when(cidx <= NCHUNK - 4)
            def _():
                issue_src(off, cidx + 3)
            wait_weights(off, cidx)
            _scale(rows[off], wbs[off])

            @pl.when(cidx >= 1)
            def _():
                wait_scatter(nbuf)

            @pl.when(cidx <= NCHUNK - 3)
            def _():
                wait_src(nbuf, cidx + 2)
                issue_gather(nbuf)
                issue_weights(nbuf, cidx + 2)
                issue_dst(nbuf, cidx + 2)
            wait_dst(off, cidx)
            issue_scatter(off)
        return carry

    lax.fori_loop(0, NCHUNK // NBUF, slot_body, 0)

    wait_gather(0)
    wait_weights(0, NCHUNK - 2)
    _scale(rows[0], wbs[0])
    wait_scatter(2)
    wait_dst(0, NCHUNK - 2)
    issue_scatter(0)

    wait_gather(1)
    wait_weights(1, NCHUNK - 1)
    _scale(rows[1], wbs[1])
    wait_scatter(0)
    wait_dst(1, NCHUNK - 1)
    issue_scatter(1)

    wait_scatter(1)

    plsc.subcore_barrier()
    for k in range(WB_ROUNDS):
        cidx = s + k * NS

        @pl.when(cidx < NWB)
        def _():
            base = cidx * WB_CHUNK
            pltpu.sync_copy(acc_sh.at[pl.ds(base, WB_CHUNK)], rows0)
            pltpu.sync_copy(rows0, out_hbm.at[c, pl.ds(base, WB_CHUNK)])


def _make_segsum(interpret=False):
    mesh = plsc.VectorSubcoreMesh(core_axis_name="c", subcore_axis_name="s",
                                  num_cores=NC, num_subcores=NS)
    return pl.kernel(
        _sc_segsum_body,
        out_type=jax.ShapeDtypeStruct((NC, N, D), jnp.float32),
        mesh=mesh,
        scratch_types=(
            [pltpu.VMEM_SHARED((N, D), jnp.float32)]
            + [pltpu.VMEM((CHUNK, D), jnp.float32) for _ in range(NBUF)]
            + [pltpu.VMEM((1, CHUNK), jnp.int32) for _ in range(2 * NBUF)]
            + [pltpu.VMEM((16, CHUNK), jnp.float32) for _ in range(NBUF)]
            + [pltpu.SemaphoreType.DMA for _ in range(5 * NBUF)]
        ),
        interpret=interpret,
    )


def _make_combo(edge_index, edge_weight):
    src = edge_index[0].reshape(NW, NCHUNK, 1, CHUNK)
    dst = edge_index[1].reshape(NW, NCHUNK, 1, CHUNK)
    w5 = edge_weight.reshape(NW, NCHUNK, CHUNK // 16, 16)
    wt = jnp.transpose(w5, (0, 1, 3, 2))[..., None]
    w16 = jnp.broadcast_to(
        wt, (NW, NCHUNK, 16, CHUNK // 16, 16)).reshape(NW, NCHUNK, 16, CHUNK)
    return src, dst, w16


def _tc_layer_body(p_ref, h_ref, wrelT_ref, wrootT_ref, b_ref,
                   gamma_ref, beta_ref, out_ref):
    agg = p_ref[0] + p_ref[1]
    y = (jnp.dot(agg, wrelT_ref[...], preferred_element_type=jnp.float32)
         + jnp.dot(h_ref[...], wrootT_ref[...], preferred_element_type=jnp.float32)
         + b_ref[...])
    mean = jnp.mean(y, axis=0, keepdims=True)
    var = jnp.mean(jnp.square(y - mean), axis=0, keepdims=True)
    yn = (y - mean) * lax.rsqrt(var + 1e-5) * gamma_ref[...] + beta_ref[...]
    out_ref[...] = jnp.maximum(yn, 0.0)


def _make_tc_layer(interpret=False):
    return pl.pallas_call(
        _tc_layer_body,
        out_shape=jax.ShapeDtypeStruct((N, D), jnp.float32),
        interpret=interpret,
    )


def _tc_final_body(p_ref, h_ref, batch_ref, wrelT_ref, wrootT_ref, b_ref,
                   wlinT_ref, blin_ref, out_ref):
    agg = p_ref[0] + p_ref[1]
    y = (jnp.dot(agg, wrelT_ref[...], preferred_element_type=jnp.float32)
         + jnp.dot(h_ref[...], wrootT_ref[...], preferred_element_type=jnp.float32)
         + b_ref[...])
    gid = lax.broadcasted_iota(jnp.int32, (G, N), 0)
    sel = (batch_ref[...] == gid).astype(jnp.float32)
    sums = jnp.dot(sel, y, preferred_element_type=jnp.float32)
    counts = jnp.sum(sel, axis=1, keepdims=True)
    pooled = sums / jnp.maximum(counts, 1.0)
    out_ref[...] = (jnp.dot(pooled, wlinT_ref[...],
                            preferred_element_type=jnp.float32)
                    + blin_ref[...])


def _make_tc_final(interpret=False):
    return pl.pallas_call(
        _tc_final_body,
        out_shape=jax.ShapeDtypeStruct((G, 2), jnp.float32),
        interpret=interpret,
    )


@jax.jit
def kernel(x, edge_index, edge_weight, batch,
           W1_rel, b1, W1_root, W2_rel, b2, W2_root, W3_rel, b3, W3_root,
           gamma1, beta1, gamma2, beta2, Wlin, blin):
    src, dst, w16 = _make_combo(edge_index, edge_weight)
    zeros = jnp.zeros((N, D), jnp.float32)

    segsum = _make_segsum()
    tc_layer = _make_tc_layer()
    tc_final = _make_tc_final()

    p1 = segsum(x, src, dst, w16, zeros)
    h1 = tc_layer(p1, x, W1_rel.T, W1_root.T, b1[None],
                  gamma1[None], beta1[None])
    p2 = segsum(h1, src, dst, w16, zeros)
    h2 = tc_layer(p2, h1, W2_rel.T, W2_root.T, b2[None],
                  gamma2[None], beta2[None])
    p3 = segsum(h2, src, dst, w16, zeros)
    return tc_final(p3, h2, batch[None], W3_rel.T, W3_root.T, b3[None],
                    Wlin.T, blin[None])


# speedup vs baseline: 8.3488x; 1.0012x over previous
import jax
import jax.numpy as jnp
from jax import lax
from jax.experimental import pallas as pl
from jax.experimental.pallas import tpu as pltpu
from jax.experimental.pallas import tpu_sc as plsc

N = 10000
D = 128
E = 320000
G = 64

NC = 2
NS = 16
NW = NC * NS
EPT = E // NW
CHUNK = 80
NCHUNK = EPT // CHUNK
WB_CHUNK = 80
NWB = N // WB_CHUNK
WB_ROUNDS = (NWB + NS - 1) // NS
FB = D // 16
NBUF = 3


def _scale(rows_v, w_v):
    def body(eb, carry):
        col = pl.multiple_of(eb * 16, 16)
        for k in range(16):
            e = eb * 16 + k
            wvec = w_v[k, pl.ds(col, 16)]
            for f in range(FB):
                rows_v[e, pl.ds(f * 16, 16)] = (
                    rows_v[e, pl.ds(f * 16, 16)] * wvec)
        return carry
    lax.fori_loop(0, CHUNK // 16, body, 0)


def _sc_segsum_body(x_hbm, src_hbm, dst_hbm, w16_hbm, zeros_hbm, out_hbm,
                    acc_sh, rows0, rows1, rows2, sb0, sb1, sb2,
                    db0, db1, db2, wb0, wb1, wb2,
                    g0, g1, g2, s0, s1, s2, ws0, ws1, ws2,
                    is0, is1, is2, id0, id1, id2):
    c = lax.axis_index("c")
    s = lax.axis_index("s")
    wid = c * NS + s
    rows = (rows0, rows1, rows2)
    sbs = (sb0, sb1, sb2)
    dbs = (db0, db1, db2)
    wbs = (wb0, wb1, wb2)
    gsems = (g0, g1, g2)
    ssems = (s0, s1, s2)
    wsems = (ws0, ws1, ws2)
    srcsems = (is0, is1, is2)
    dstsems = (id0, id1, id2)

    for k in range(WB_ROUNDS):
        cidx = s + k * NS

        @pl.when(cidx < NWB)
        def _():
            pltpu.sync_copy(zeros_hbm.at[pl.ds(cidx * WB_CHUNK, WB_CHUNK)],
                            acc_sh.at[pl.ds(cidx * WB_CHUNK, WB_CHUNK)])
    plsc.subcore_barrier()

    def issue_src(buf, chunk_idx):
        pltpu.async_copy(src_hbm.at[wid, chunk_idx], sbs[buf], srcsems[buf])

    def wait_src(buf, chunk_idx):
        pltpu.make_async_copy(src_hbm.at[wid, chunk_idx], sbs[buf],
                              srcsems[buf]).wait()

    def issue_dst(buf, chunk_idx):
        pltpu.async_copy(dst_hbm.at[wid, chunk_idx], dbs[buf], dstsems[buf])

    def wait_dst(buf, chunk_idx):
        pltpu.make_async_copy(dst_hbm.at[wid, chunk_idx], dbs[buf],
                              dstsems[buf]).wait()

    def issue_weights(buf, chunk_idx):
        pltpu.async_copy(w16_hbm.at[wid, chunk_idx], wbs[buf], wsems[buf])

    def wait_weights(buf, chunk_idx):
        pltpu.make_async_copy(w16_hbm.at[wid, chunk_idx], wbs[buf],
                              wsems[buf]).wait()

    def issue_gather(buf):
        pltpu.async_copy(x_hbm.at[sbs[buf].at[0]], rows[buf], gsems[buf])

    def wait_gather(buf):
        pltpu.make_async_copy(x_hbm.at[sbs[buf].at[0]], rows[buf],
                              gsems[buf]).wait()

    def issue_scatter(buf):
        pltpu.async_copy(rows[buf], acc_sh.at[dbs[buf].at[0]], ssems[buf],
                         add=True)

    def wait_scatter(buf):
        pltpu.make_async_copy(rows[buf], acc_sh.at[dbs[buf].at[0]],
                              ssems[buf]).wait()

    pltpu.sync_copy(src_hbm.at[wid, 0], sbs[0])
    issue_gather(0)
    issue_weights(0, 0)
    issue_dst(0, 0)
    pltpu.sync_copy(src_hbm.at[wid, 1], sbs[1])
    issue_gather(1)
    issue_weights(1, 1)
    issue_dst(1, 1)
    issue_src(2, 2)

    def slot_body(g, carry):
        for off in range(NBUF):
            cidx = 3 * g + off
            nbuf = (off + 2) % 3
            wait_gather(off)

            @pl.when(cidx <= NCHUNK - 4)
            def _():
                issue_src(off, cidx + 3)
            wait_weights(off, cidx)
            _scale(rows[off], wbs[off])

            @pl.when(cidx >= 1)
            def _():
                wait_scatter(nbuf)

            @pl.when(cidx <= NCHUNK - 3)
            def _():
                wait_src(nbuf, cidx + 2)
                issue_gather(nbuf)
                issue_weights(nbuf, cidx + 2)
                issue_dst(nbuf, cidx + 2)
            wait_dst(off, cidx)
            issue_scatter(off)
        return carry

    lax.fori_loop(0, NCHUNK // NBUF, slot_body, 0)

    wait_gather(0)
    wait_weights(0, NCHUNK - 2)
    _scale(rows[0], wbs[0])
    wait_scatter(2)
    wait_dst(0, NCHUNK - 2)
    issue_scatter(0)

    wait_gather(1)
    wait_weights(1, NCHUNK - 1)
    _scale(rows[1], wbs[1])
    wait_scatter(0)
    wait_dst(1, NCHUNK - 1)
    issue_scatter(1)

    wait_scatter(1)

    plsc.subcore_barrier()
    for k in range(WB_ROUNDS):
        cidx = s + k * NS

        @pl.when(cidx < NWB)
        def _():
            base = cidx * WB_CHUNK
            pltpu.sync_copy(acc_sh.at[pl.ds(base, WB_CHUNK)], rows0)
            pltpu.sync_copy(rows0, out_hbm.at[c, pl.ds(base, WB_CHUNK)])


def _make_segsum(interpret=False):
    mesh = plsc.VectorSubcoreMesh(core_axis_name="c", subcore_axis_name="s",
                                  num_cores=NC, num_subcores=NS)
    return pl.kernel(
        _sc_segsum_body,
        out_type=jax.ShapeDtypeStruct((NC, N, D), jnp.float32),
        mesh=mesh,
        scratch_types=(
            [pltpu.VMEM_SHARED((N, D), jnp.float32)]
            + [pltpu.VMEM((CHUNK, D), jnp.float32) for _ in range(NBUF)]
            + [pltpu.VMEM((1, CHUNK), jnp.int32) for _ in range(2 * NBUF)]
            + [pltpu.VMEM((16, CHUNK), jnp.float32) for _ in range(NBUF)]
            + [pltpu.SemaphoreType.DMA for _ in range(5 * NBUF)]
        ),
        interpret=interpret,
    )


def _make_combo(edge_index, edge_weight):
    src = edge_index[0].reshape(NW, NCHUNK, 1, CHUNK)
    dst = edge_index[1].reshape(NW, NCHUNK, 1, CHUNK)
    w5 = edge_weight.reshape(NW, NCHUNK, CHUNK // 16, 16)
    wt = jnp.transpose(w5, (0, 1, 3, 2))[..., None]
    w16 = jnp.broadcast_to(
        wt, (NW, NCHUNK, 16, CHUNK // 16, 16)).reshape(NW, NCHUNK, 16, CHUNK)
    return src, dst, w16


def _tc_layer_body(p_ref, h_ref, wrelT_ref, wrootT_ref, b_ref,
                   gamma_ref, beta_ref, out_ref):
    agg = p_ref[0] + p_ref[1]
    y = (jnp.dot(agg, wrelT_ref[...], preferred_element_type=jnp.float32)
         + jnp.dot(h_ref[...], wrootT_ref[...], preferred_element_type=jnp.float32)
         + b_ref[...])
    mean = jnp.mean(y, axis=0, keepdims=True)
    var = jnp.mean(jnp.square(y - mean), axis=0, keepdims=True)
    yn = (y - mean) * lax.rsqrt(var + 1e-5) * gamma_ref[...] + beta_ref[...]
    out_ref[...] = jnp.maximum(yn, 0.0)


def _make_tc_layer(interpret=False):
    return pl.pallas_call(
        _tc_layer_body,
        out_shape=jax.ShapeDtypeStruct((N, D), jnp.float32),
        interpret=interpret,
    )


def _tc_final_body(p_ref, h_ref, batch_ref, wrelT_ref, wrootT_ref, b_ref,
                   wlinT_ref, blin_ref, out_ref):
    agg = p_ref[0] + p_ref[1]
    y = (jnp.dot(agg, wrelT_ref[...], preferred_element_type=jnp.float32)
         + jnp.dot(h_ref[...], wrootT_ref[...], preferred_element_type=jnp.float32)
         + b_ref[...])
    gid = lax.broadcasted_iota(jnp.int32, (G, N), 0)
    sel = (batch_ref[...] == gid).astype(jnp.float32)
    sums = jnp.dot(sel, y, preferred_element_type=jnp.float32)
    counts = jnp.sum(sel, axis=1, keepdims=True)
    pooled = sums / jnp.maximum(counts, 1.0)
    out_ref[...] = (jnp.dot(pooled, wlinT_ref[...],
                            preferred_element_type=jnp.float32)
                    + blin_ref[...])


def _make_tc_final(interpret=False):
    return pl.pallas_call(
        _tc_final_body,
        out_shape=jax.ShapeDtypeStruct((G, 2), jnp.float32),
        interpret=interpret,
    )


@jax.jit
def kernel(x, edge_index, edge_weight, batch,
           W1_rel, b1, W1_root, W2_rel, b2, W2_root, W3_rel, b3, W3_root,
           gamma1, beta1, gamma2, beta2, Wlin, blin):
    src, dst, w16 = _make_combo(edge_index, edge_weight)
    zeros = jnp.zeros((N, D), jnp.float32)

    segsum = _make_segsum()
    tc_layer = _make_tc_layer()
    tc_final = _make_tc_final()

    p1 = segsum(x, src, dst, w16, zeros)
    h1 = tc_layer(p1, x, W1_rel.T, W1_root.T, b1[None],
                  gamma1[None], beta1[None])
    p2 = segsum(h1, src, dst, w16, zeros)
    h2 = tc_layer(p2, h1, W2_rel.T, W2_root.T, b2[None],
                  gamma2[None], beta2[None])
    p3 = segsum(h2, src, dst, w16, zeros)
    return tc_final(p3, h2, batch[None], W3_rel.T, W3_root.T, b3[None],
                    Wlin.T, blin[None])
